# Initial kernel scaffold; baseline (speedup 1.0000x reference)
import jax
import jax.numpy as jnp
from jax.experimental import pallas as pl


def kernel(poses, cam_poses, embeddings, query, nn_k):
    raise NotImplementedError("write your pallas kernel here")


# speedup vs baseline: 1.8964x; 1.8964x over previous
import functools

import jax
import jax.numpy as jnp
from jax import lax
from jax.experimental import pallas as pl
from jax.experimental.pallas import tpu as pltpu
from jax.experimental.pallas import tpu_sc as plsc

_K = 8
_CHUNK = 1024
_BIGF = 3e38
_IBIG = 2147483647


def _logmap_from_matrix(R):
    trace = R[:, 0, 0] + R[:, 1, 1] + R[:, 2, 2]
    cos_t = jnp.clip((trace - 1.0) * 0.5, -1.0 + 1e-7, 1.0 - 1e-7)
    theta = jnp.arccos(cos_t)
    sin_t = jnp.sin(theta)
    scale = jnp.where(theta < 1e-6, 0.5, theta / (2.0 * sin_t))
    vee = jnp.stack([R[:, 2, 1] - R[:, 1, 2], R[:, 0, 2] - R[:, 2, 0],
                     R[:, 1, 0] - R[:, 0, 1]], axis=-1)
    return scale[:, None] * vee


def _feat6(poses, w=0.01):
    return jnp.concatenate(
        [(1.0 - w) * poses[:, :3, 3], w * _logmap_from_matrix(poses[:, :3, :3])],
        axis=1)


def _topk_body(qf_ref, tf_ref, q2_ref, t2_ref, oi_ref, tv, ti):
    c = pl.program_id(0)
    nch = pl.num_programs(0)
    q = qf_ref.shape[0]
    chunk = tf_ref.shape[1]

    @pl.when(c == 0)
    def _init():
        tv[...] = jnp.full((q, _K), _BIGF, jnp.float32)
        ti[...] = jnp.zeros((q, _K), jnp.int32)

    mm = lax.dot_general(qf_ref[...], tf_ref[...],
                         (((1,), (0,)), ((), ())),
                         preferred_element_type=jnp.float32)
    d2 = (q2_ref[...] + t2_ref[...]) - 2.0 * mm
    col = lax.broadcasted_iota(jnp.int32, (q, chunk), 1) + c * chunk

    evs, eis = [], []
    for _ in range(_K):
        v = jnp.min(d2, axis=1, keepdims=True)
        i = jnp.min(jnp.where(d2 == v, col, _IBIG), axis=1, keepdims=True)
        evs.append(v)
        eis.append(i)
        d2 = jnp.where(col == i, _BIGF, d2)
    ev = jnp.concatenate(evs, axis=1)
    ei = jnp.concatenate(eis, axis=1)

    cv = jnp.concatenate([tv[...], ev], axis=1)
    ci = jnp.concatenate([ti[...], ei], axis=1)
    nvs, nis = [], []
    for _ in range(_K):
        v = jnp.min(cv, axis=1, keepdims=True)
        i = jnp.min(jnp.where(cv == v, ci, _IBIG), axis=1, keepdims=True)
        nvs.append(v)
        nis.append(i)
        cv = jnp.where(ci == i, _BIGF, cv)
    tv[...] = jnp.concatenate(nvs, axis=1)
    ti[...] = jnp.concatenate(nis, axis=1)

    @pl.when(c == nch - 1)
    def _emit():
        oi_ref[...] = ti[...]


def _topk_indices(qfp, tft, q2c, t2p):
    q = qfp.shape[0]
    npad = tft.shape[1]
    nch = npad // _CHUNK
    return pl.pallas_call(
        _topk_body,
        grid=(nch,),
        in_specs=[
            pl.BlockSpec((q, 8), lambda c: (0, 0)),
            pl.BlockSpec((8, _CHUNK), lambda c: (0, c)),
            pl.BlockSpec((q, 1), lambda c: (0, 0)),
            pl.BlockSpec((1, _CHUNK), lambda c: (0, c)),
        ],
        out_specs=pl.BlockSpec((q, _K), lambda c: (0, 0)),
        out_shape=jax.ShapeDtypeStruct((q, _K), jnp.int32),
        scratch_shapes=[
            pltpu.VMEM((q, _K), jnp.float32),
            pltpu.VMEM((q, _K), jnp.int32),
        ],
    )(qfp, tft, q2c, t2p)


def _gather_rows(idx_flat, t0, t1, t2):
    b = idx_flat.shape[0]
    nw = 32
    bw = b // nw
    mesh = plsc.VectorSubcoreMesh(core_axis_name="c", subcore_axis_name="s")
    row = jax.ShapeDtypeStruct((b, 16), jnp.float32)

    @functools.partial(
        pl.kernel, mesh=mesh,
        out_type=[row, row, row],
        compiler_params=pltpu.CompilerParams(use_tc_tiling_on_sc=False),
        scratch_types=[
            pltpu.VMEM((bw,), jnp.int32),
            pltpu.VMEM((bw, 16), jnp.float32),
            pltpu.VMEM((bw, 16), jnp.float32),
            pltpu.VMEM((bw, 16), jnp.float32),
            pltpu.SemaphoreType.DMA,
            pltpu.SemaphoreType.DMA,
            pltpu.SemaphoreType.DMA,
        ],
    )
    def k(idx_hbm, t0_hbm, t1_hbm, t2_hbm, o0, o1, o2,
          idx_v, r0, r1, r2, s0, s1, s2):
        wid = lax.axis_index("s") * 2 + lax.axis_index("c")
        base = wid * bw
        pltpu.sync_copy(idx_hbm.at[pl.ds(base, bw)], idx_v)
        c0 = pltpu.async_copy(t0_hbm.at[idx_v], r0, s0)
        c1 = pltpu.async_copy(t1_hbm.at[idx_v], r1, s1)
        c2 = pltpu.async_copy(t2_hbm.at[idx_v], r2, s2)
        c0.wait()
        c1.wait()
        c2.wait()
        pltpu.sync_copy(r0, o0.at[pl.ds(base, bw)])
        pltpu.sync_copy(r1, o1.at[pl.ds(base, bw)])
        pltpu.sync_copy(r2, o2.at[pl.ds(base, bw)])

    return k(idx_flat, t0, t1, t2)


def kernel(poses, cam_poses, embeddings, query, nn_k):
    n = poses.shape[0]
    q = query.shape[0]
    tf = _feat6(poses)
    qf = _feat6(query)
    q2 = (qf ** 2).sum(-1)
    t2 = (tf ** 2).sum(-1)

    npad = ((n + _CHUNK - 1) // _CHUNK) * _CHUNK
    qfp = jnp.zeros((q, 8), jnp.float32).at[:, :6].set(qf)
    tft = jnp.zeros((8, npad), jnp.float32).at[:6, :n].set(tf.T)
    q2c = q2[:, None]
    t2p = jnp.full((1, npad), 1e30, jnp.float32).at[0, :n].set(t2)

    idx = _topk_indices(qfp, tft, q2c, t2p)
    flat = idx.reshape(-1)

    p_rows, c_rows, e_rows = _gather_rows(
        flat,
        poses.reshape(n, 16),
        cam_poses.reshape(n, 16),
        embeddings,
    )
    return (p_rows.reshape(q, _K, 4, 4),
            c_rows.reshape(q, _K, 4, 4),
            e_rows.reshape(q, _K, 16))


# speedup vs baseline: 2.8194x; 1.4867x over previous
import functools

import jax
import jax.numpy as jnp
from jax import lax
from jax.experimental import pallas as pl
from jax.experimental.pallas import tpu as pltpu
from jax.experimental.pallas import tpu_sc as plsc

_K = 8
_CHUNK = 2048
_BIGF = 3e38


def _logmap_from_matrix(R):
    trace = R[:, 0, 0] + R[:, 1, 1] + R[:, 2, 2]
    cos_t = jnp.clip((trace - 1.0) * 0.5, -1.0 + 1e-7, 1.0 - 1e-7)
    theta = jnp.arccos(cos_t)
    sin_t = jnp.sin(theta)
    scale = jnp.where(theta < 1e-6, 0.5, theta / (2.0 * sin_t))
    vee = jnp.stack([R[:, 2, 1] - R[:, 1, 2], R[:, 0, 2] - R[:, 2, 0],
                     R[:, 1, 0] - R[:, 0, 1]], axis=-1)
    return scale[:, None] * vee


def _feat6(poses, w=0.01):
    return jnp.concatenate(
        [(1.0 - w) * poses[:, :3, 3], w * _logmap_from_matrix(poses[:, :3, :3])],
        axis=1)


def _topk_body(qf_ref, tf_ref, q2_ref, t2_ref, col_ref, oi_ref, tv, ti):
    c = pl.program_id(0)
    nch = pl.num_programs(0)
    q = qf_ref.shape[0]
    chunk = tf_ref.shape[1]

    @pl.when(c == 0)
    def _init():
        tv[...] = jnp.full((q, _K), _BIGF, jnp.float32)
        ti[...] = jnp.zeros((q, _K), jnp.float32)

    mm = lax.dot_general(qf_ref[...], tf_ref[...],
                         (((1,), (0,)), ((), ())),
                         preferred_element_type=jnp.float32)
    d2 = (q2_ref[...] + t2_ref[...]) - 2.0 * mm
    col = col_ref[...]

    wv = tv[...]
    wi = ti[...]
    nvs, nis = [], []
    for _ in range(_K):
        v = jnp.minimum(jnp.min(d2, axis=1, keepdims=True),
                        jnp.min(wv, axis=1, keepdims=True))
        i = jnp.minimum(
            jnp.min(jnp.where(d2 == v, col, _BIGF), axis=1, keepdims=True),
            jnp.min(jnp.where(wv == v, wi, _BIGF), axis=1, keepdims=True))
        nvs.append(v)
        nis.append(i)
        d2 = jnp.where(col == i, _BIGF, d2)
        wv = jnp.where(wi == i, _BIGF, wv)
    tv[...] = jnp.concatenate(nvs, axis=1)
    ti[...] = jnp.concatenate(nis, axis=1)

    @pl.when(c == nch - 1)
    def _emit():
        oi_ref[...] = ti[...].astype(jnp.int32)


def _topk_indices(qfp, tft, q2c, t2p, colp):
    q = qfp.shape[0]
    npad = tft.shape[1]
    nch = npad // _CHUNK
    return pl.pallas_call(
        _topk_body,
        grid=(nch,),
        in_specs=[
            pl.BlockSpec((q, 8), lambda c: (0, 0)),
            pl.BlockSpec((8, _CHUNK), lambda c: (0, c)),
            pl.BlockSpec((q, 1), lambda c: (0, 0)),
            pl.BlockSpec((1, _CHUNK), lambda c: (0, c)),
            pl.BlockSpec((1, _CHUNK), lambda c: (0, c)),
        ],
        out_specs=pl.BlockSpec((q, _K), lambda c: (0, 0)),
        out_shape=jax.ShapeDtypeStruct((q, _K), jnp.int32),
        scratch_shapes=[
            pltpu.VMEM((q, _K), jnp.float32),
            pltpu.VMEM((q, _K), jnp.float32),
        ],
    )(qfp, tft, q2c, t2p, colp)


def _gather_rows(idx_flat, t0, t1, t2):
    b = idx_flat.shape[0]
    nw = 32
    bw = b // nw
    mesh = plsc.VectorSubcoreMesh(core_axis_name="c", subcore_axis_name="s")

    @functools.partial(
        pl.kernel, mesh=mesh,
        out_type=[jax.ShapeDtypeStruct((b, 16), jnp.float32),
                  jax.ShapeDtypeStruct((b, 16), jnp.float32),
                  jax.ShapeDtypeStruct((b, 16), jnp.float32)],
        compiler_params=pltpu.CompilerParams(use_tc_tiling_on_sc=False),
        scratch_types=[
            pltpu.VMEM((bw,), jnp.int32),
            pltpu.VMEM((bw, 16), jnp.float32),
            pltpu.VMEM((bw, 16), jnp.float32),
            pltpu.VMEM((bw, 16), jnp.float32),
            pltpu.SemaphoreType.DMA,
            pltpu.SemaphoreType.DMA,
            pltpu.SemaphoreType.DMA,
        ],
    )
    def k(idx_hbm, t0_hbm, t1_hbm, t2_hbm, o0, o1, o2,
          idx_v, r0, r1, r2, s0, s1, s2):
        wid = lax.axis_index("s") * 2 + lax.axis_index("c")
        base = wid * bw
        pltpu.sync_copy(idx_hbm.at[pl.ds(base, bw)], idx_v)
        c0 = pltpu.async_copy(t0_hbm.at[idx_v], r0, s0)
        c1 = pltpu.async_copy(t1_hbm.at[idx_v], r1, s1)
        c2 = pltpu.async_copy(t2_hbm.at[idx_v], r2, s2)
        c0.wait()
        c1.wait()
        c2.wait()
        pltpu.sync_copy(r0, o0.at[pl.ds(base, bw)])
        pltpu.sync_copy(r1, o1.at[pl.ds(base, bw)])
        pltpu.sync_copy(r2, o2.at[pl.ds(base, bw)])

    return k(idx_flat, t0, t1, t2)


def kernel(poses, cam_poses, embeddings, query, nn_k):
    n = poses.shape[0]
    q = query.shape[0]
    tf = _feat6(poses)
    qf = _feat6(query)
    q2 = (qf ** 2).sum(-1)
    t2 = (tf ** 2).sum(-1)

    npad = ((n + _CHUNK - 1) // _CHUNK) * _CHUNK
    qfp = jnp.zeros((q, 8), jnp.float32).at[:, :6].set(qf)
    tft = jnp.zeros((8, npad), jnp.float32).at[:6, :n].set(tf.T)
    q2c = q2[:, None]
    t2p = jnp.full((1, npad), 1e30, jnp.float32).at[0, :n].set(t2)
    colp = jnp.arange(npad, dtype=jnp.float32)[None, :]

    idx = _topk_indices(qfp, tft, q2c, t2p, colp)
    flat = idx.reshape(-1)

    p_rows, c_rows, e_rows = _gather_rows(
        flat, poses.reshape(n, 16), cam_poses.reshape(n, 16), embeddings)
    return (p_rows.reshape(q, _K, 4, 4),
            c_rows.reshape(q, _K, 4, 4),
            e_rows.reshape(q, _K, 16))
